# baseline (speedup 1.0000x reference)
import functools

import jax
import jax.numpy as jnp
from jax import lax
from jax.experimental import pallas as pl
from jax.experimental.pallas import tpu as pltpu
from jax.experimental.pallas import tpu_sc as plsc

NUM_EMBEDDINGS = 1000000
D = 64
BATCH = 4096
HIST = 200

_info = plsc.get_sparse_core_info()
NC = _info.num_cores
NS = _info.num_subcores
NW = NC * NS
L = _info.num_lanes

C = BATCH // NW
NQ = C // L

NKFULL = NUM_EMBEDDINGS // 128
NTAIL = NUM_EMBEDDINGS - NKFULL * 128
W2ROWS = NUM_EMBEDDINGS // 2
NK = NKFULL + 1
W2PAD = NK * D


def _retile_block(b_ref, o_ref):
    x = jnp.transpose(b_ref[...]).reshape(D, 2, D)
    o_ref[...] = jnp.concatenate([x[:, 0, :], x[:, 1, :]], axis=-1)


_retile = pl.pallas_call(
    _retile_block,
    grid=(NK,),
    in_specs=[pl.BlockSpec((D, 128), lambda k: (0, k))],
    out_specs=pl.BlockSpec((D, 2 * D), lambda k: (k, 0)),
    out_shape=jax.ShapeDtypeStruct((W2PAD, 2 * D), jnp.float32),
)


@functools.partial(
    pl.kernel,
    out_type=jax.ShapeDtypeStruct((HIST, D // 8, BATCH // C, 8, C),
                                  jnp.float32),
    mesh=plsc.VectorSubcoreMesh(core_axis_name="c", subcore_axis_name="s"),
    scratch_types=[
        pltpu.VMEM((HIST, C), jnp.int32),
        pltpu.VMEM((2, C, D), jnp.float32),
        pltpu.VMEM((2, D // 8, 8, C), jnp.float32),
        pltpu.SemaphoreType.DMA((2,)),
        pltpu.SemaphoreType.DMA((2,)),
    ],
    compiler_params=pltpu.CompilerParams(
        use_tc_tiling_on_sc=False, needs_layout_passes=False),
)
def _emb_lookup(ids_hbm, table_hbm, out_hbm, r_v, g_v, o_v, gsem, wsem):
    wid = lax.axis_index("s") * NC + lax.axis_index("c")
    col0 = wid * C

    pltpu.sync_copy(ids_hbm.at[:, pl.ds(col0, C)], r_v)

    def gather(u, b):
        return pltpu.make_async_copy(
            table_hbm.at[r_v.at[u]], g_v.at[b], gsem.at[b])

    def write(u, b):
        return pltpu.make_async_copy(
            o_v.at[b], out_hbm.at[u, :, wid], wsem.at[b])

    lanes = lax.iota(jnp.int32, L)

    def transpose_unit(b):
        src = g_v.at[b]
        dst = o_v.at[b]

        def dloop(dd, carry):
            d2 = (dd + lanes) & (D - 1)
            dhi = lax.shift_right_logical(d2, 3)
            dlo = d2 & 7
            for q in range(NQ):
                cvec = lanes + L * q
                val = plsc.load_gather(src, [cvec, d2])
                plsc.store_scatter(dst, [dhi, dlo, cvec], val)
            return carry

        lax.fori_loop(0, D, dloop, 0)

    gather(0, 0).start()
    gather(1, 1).start()

    def outer(p, carry):
        for b in range(2):
            u = 2 * p + b

            @pl.when(p > 0)
            def _():
                write(u - 2, b).wait()
            gather(u, b).wait()
            transpose_unit(b)
            write(u, b).start()

            @pl.when(p < HIST // 2 - 1)
            def _():
                gather(u + 2, b).start()
        return carry

    lax.fori_loop(0, HIST // 2, outer, 0)
    write(HIST - 2, 0).wait()
    write(HIST - 1, 1).wait()


def kernel(token_ids, weight):
    ids_t = token_ids.T.astype(jnp.int32)
    w2 = _retile(weight.T)
    table = w2.reshape(2 * W2PAD, D)
    out5 = _emb_lookup(ids_t, table)
    return out5.transpose(2, 4, 0, 1, 3).reshape(BATCH, HIST, D)


# speedup vs baseline: 4.8793x; 4.8793x over previous
import functools

import jax
import jax.numpy as jnp
from jax import lax
from jax.experimental import pallas as pl
from jax.experimental.pallas import tpu as pltpu
from jax.experimental.pallas import tpu_sc as plsc

NUM_EMBEDDINGS = 1000000
D = 64
BATCH = 4096
HIST = 200

_info = plsc.get_sparse_core_info()
NC = _info.num_cores
NS = _info.num_subcores
NW = NC * NS
L = _info.num_lanes

C = BATCH // NW
NQ = C // L

NKFULL = NUM_EMBEDDINGS // 128
NTAIL = NUM_EMBEDDINGS - NKFULL * 128
W2ROWS = NUM_EMBEDDINGS // 2
NB = 4


@functools.partial(
    pl.kernel,
    out_type=jax.ShapeDtypeStruct((W2ROWS, 2 * D), jnp.float32),
    mesh=plsc.VectorSubcoreMesh(core_axis_name="c", subcore_axis_name="s"),
    scratch_types=[
        pltpu.VMEM((NB, D, 128), jnp.float32),
        pltpu.VMEM((NB, D, 128), jnp.float32),
        pltpu.SemaphoreType.DMA((NB,)),
        pltpu.SemaphoreType.DMA((NB,)),
    ],
    compiler_params=pltpu.CompilerParams(
        use_tc_tiling_on_sc=True, needs_layout_passes=False),
)
def _retile(wt_hbm, wtail_hbm, w2_hbm, b_v, o_v, gsem, wsem):
    wid = lax.axis_index("s") * NC + lax.axis_index("c")
    nk = jnp.where(wid < NKFULL % NW, NKFULL // NW + 1, NKFULL // NW)

    def load(j, b):
        k = wid + NW * j
        return pltpu.make_async_copy(
            wt_hbm.at[:, pl.ds(128 * k, 128)], b_v.at[b], gsem.at[b])

    def write(j, b):
        k = wid + NW * j
        return pltpu.make_async_copy(
            o_v.at[b], w2_hbm.at[pl.ds(D * k, D)], wsem.at[b])

    lanes = lax.iota(jnp.int32, L)

    def transpose_block(b):
        src = b_v.at[b]
        dst = o_v.at[b]

        def rloop(r, carry):
            r2 = (r + lanes) & (D - 1)
            i2 = r2 + r2
            for q in range(NQ):
                dvec = (L * q) % D + lanes
                val = plsc.load_gather(src, [dvec, i2 + (q // (NQ // 2))])
                plsc.store_scatter(dst, [r2, lanes + L * q], val)
            return carry

        lax.fori_loop(0, D, rloop, 0)

    for i in range(NB - 1):
        @pl.when(nk > i)
        def _():
            load(i, i).start()

    def body(p, carry):
        for b in range(NB):
            j = NB * p + b

            @pl.when(j < nk)
            def _():
                @pl.when(j >= NB)
                def _():
                    write(j - NB, b).wait()
                load(j, b).wait()
                transpose_block(b)
                write(j, b).start()

                @pl.when(j + NB - 1 < nk)
                def _():
                    load(j + NB - 1, (b + NB - 1) % NB).start()
        return carry

    lax.fori_loop(0, (NKFULL // NW + 1 + NB - 1) // NB, body, 0)

    for b in range(NB):
        write(0, b).wait()

    @pl.when(wid == NW - 1)
    def _():
        pltpu.sync_copy(wtail_hbm, b_v.at[0])
        transpose_block(0)
        pltpu.sync_copy(o_v.at[0, pl.ds(0, NTAIL // 2)],
                        w2_hbm.at[pl.ds(D * NKFULL, NTAIL // 2)])


@functools.partial(
    pl.kernel,
    out_type=jax.ShapeDtypeStruct((HIST, D // 8, BATCH // C, 8, C),
                                  jnp.float32),
    mesh=plsc.VectorSubcoreMesh(core_axis_name="c", subcore_axis_name="s"),
    scratch_types=[
        pltpu.VMEM((HIST, C), jnp.int32),
        pltpu.VMEM((NB, C, D), jnp.float32),
        pltpu.VMEM((NB, D // 8, 8, C), jnp.float32),
        pltpu.SemaphoreType.DMA((NB,)),
        pltpu.SemaphoreType.DMA((NB,)),
    ],
    compiler_params=pltpu.CompilerParams(
        use_tc_tiling_on_sc=False, needs_layout_passes=False),
)
def _emb_lookup(ids_hbm, table_hbm, out_hbm, r_v, g_v, o_v, gsem, wsem):
    wid = lax.axis_index("s") * NC + lax.axis_index("c")
    col0 = wid * C

    pltpu.sync_copy(ids_hbm.at[:, pl.ds(col0, C)], r_v)

    def gather(u, b):
        return pltpu.make_async_copy(
            table_hbm.at[r_v.at[u]], g_v.at[b], gsem.at[b])

    def write(u, b):
        return pltpu.make_async_copy(
            o_v.at[b], out_hbm.at[u, :, wid], wsem.at[b])

    lanes = lax.iota(jnp.int32, L)

    def transpose_unit(b):
        src = g_v.at[b]
        dst = o_v.at[b]

        def dloop(dd, carry):
            d2 = (dd + lanes) & (D - 1)
            dhi = lax.shift_right_logical(d2, 3)
            dlo = d2 & 7
            for q in range(NQ):
                cvec = lanes + L * q
                val = plsc.load_gather(src, [cvec, d2])
                plsc.store_scatter(dst, [dhi, dlo, cvec], val)
            return carry

        lax.fori_loop(0, D, dloop, 0)

    for i in range(NB - 1):
        gather(i, i).start()

    def outer(p, carry):
        for b in range(NB):
            u = NB * p + b

            @pl.when(p > 0)
            def _():
                write(u - NB, b).wait()
            gather(u, b).wait()
            transpose_unit(b)
            write(u, b).start()

            if b == 0:
                gather(u + NB - 1, (b + NB - 1) % NB).start()
            else:
                @pl.when(p < HIST // NB - 1)
                def _():
                    gather(u + NB - 1, (b + NB - 1) % NB).start()
        return carry

    lax.fori_loop(0, HIST // NB, outer, 0)
    for b in range(NB):
        write(HIST - NB + b, b).wait()


def kernel(token_ids, weight):
    ids_t = token_ids.T.astype(jnp.int32)
    wtail = jnp.pad(weight[NKFULL * 128:].T, ((0, 0), (0, 128 - NTAIL)))
    w2 = _retile(weight.T, wtail)
    table = w2.reshape(NUM_EMBEDDINGS, D)
    out5 = _emb_lookup(ids_t, table)
    return out5.transpose(2, 4, 0, 1, 3).reshape(BATCH, HIST, D)


# speedup vs baseline: 13.8087x; 2.8300x over previous
import functools

import jax
import jax.numpy as jnp
from jax import lax
from jax.experimental import pallas as pl
from jax.experimental.pallas import tpu as pltpu
from jax.experimental.pallas import tpu_sc as plsc

NUM_EMBEDDINGS = 1000000
D = 64
BATCH = 4096
HIST = 200

_info = plsc.get_sparse_core_info()
NC = _info.num_cores
NS = _info.num_subcores
NW = NC * NS
L = _info.num_lanes

C = BATCH // NW
NQ = C // L

NKFULL = NUM_EMBEDDINGS // 128
NTAIL = NUM_EMBEDDINGS - NKFULL * 128
W2ROWS = NUM_EMBEDDINGS // 2
NB = 4


@functools.partial(
    pl.kernel,
    out_type=jax.ShapeDtypeStruct((W2ROWS, 2 * D), jnp.float32),
    mesh=plsc.VectorSubcoreMesh(core_axis_name="c", subcore_axis_name="s"),
    scratch_types=[
        pltpu.VMEM((NB, D, 128), jnp.float32),
        pltpu.VMEM((NB, D, 128), jnp.float32),
        pltpu.SemaphoreType.DMA((NB,)),
        pltpu.SemaphoreType.DMA((NB,)),
    ],
    compiler_params=pltpu.CompilerParams(
        use_tc_tiling_on_sc=True, needs_layout_passes=False),
)
def _retile(wt_hbm, wtail_hbm, w2_hbm, b_v, o_v, gsem, wsem):
    wid = lax.axis_index("s") * NC + lax.axis_index("c")
    nk = jnp.where(wid < NKFULL % NW, NKFULL // NW + 1, NKFULL // NW)

    def load(j, b):
        k = wid + NW * j
        return pltpu.make_async_copy(
            wt_hbm.at[:, pl.ds(128 * k, 128)], b_v.at[b], gsem.at[b])

    def write(j, b):
        k = wid + NW * j
        return pltpu.make_async_copy(
            o_v.at[b], w2_hbm.at[pl.ds(D * k, D)], wsem.at[b])

    lanes = lax.iota(jnp.int32, L)

    def transpose_block(b):
        src = b_v.at[b]
        dst = o_v.at[b]

        @functools.partial(plsc.parallel_loop, 0, D, unroll=2)
        def rloop(r):
            r2 = (r + lanes) & (D - 1)
            i2 = r2 + r2
            vals = [
                plsc.load_gather(
                    src, [(L * q) % D + lanes, i2 + (q // (NQ // 2))])
                for q in range(NQ)
            ]
            for q in range(NQ):
                plsc.store_scatter(dst, [r2, lanes + L * q], vals[q])

    for i in range(NB - 1):
        @pl.when(nk > i)
        def _():
            load(i, i).start()

    def body(p, carry):
        for b in range(NB):
            j = NB * p + b

            @pl.when(j < nk)
            def _():
                @pl.when(j >= NB)
                def _():
                    write(j - NB, b).wait()
                load(j, b).wait()
                transpose_block(b)
                write(j, b).start()

                @pl.when(j + NB - 1 < nk)
                def _():
                    load(j + NB - 1, (b + NB - 1) % NB).start()
        return carry

    lax.fori_loop(0, (NKFULL // NW + 1 + NB - 1) // NB, body, 0)

    for b in range(NB):
        write(0, b).wait()

    @pl.when(wid == NW - 1)
    def _():
        pltpu.sync_copy(wtail_hbm, b_v.at[0])
        transpose_block(0)
        pltpu.sync_copy(o_v.at[0, pl.ds(0, NTAIL // 2)],
                        w2_hbm.at[pl.ds(D * NKFULL, NTAIL // 2)])


@functools.partial(
    pl.kernel,
    out_type=jax.ShapeDtypeStruct((HIST, D // 8, BATCH // C, 8, C),
                                  jnp.float32),
    mesh=plsc.VectorSubcoreMesh(core_axis_name="c", subcore_axis_name="s"),
    scratch_types=[
        pltpu.VMEM((HIST, C), jnp.int32),
        pltpu.VMEM((NB, C, D), jnp.float32),
        pltpu.VMEM((NB, D // 8, 8, C), jnp.float32),
        pltpu.SemaphoreType.DMA((NB,)),
        pltpu.SemaphoreType.DMA((NB,)),
    ],
    compiler_params=pltpu.CompilerParams(
        use_tc_tiling_on_sc=False, needs_layout_passes=False),
)
def _emb_lookup(ids_hbm, table_hbm, out_hbm, r_v, g_v, o_v, gsem, wsem):
    wid = lax.axis_index("s") * NC + lax.axis_index("c")
    col0 = wid * C

    pltpu.sync_copy(ids_hbm.at[:, pl.ds(col0, C)], r_v)

    def gather(u, b):
        return pltpu.make_async_copy(
            table_hbm.at[r_v.at[u]], g_v.at[b], gsem.at[b])

    def write(u, b):
        return pltpu.make_async_copy(
            o_v.at[b], out_hbm.at[u, :, wid], wsem.at[b])

    lanes = lax.iota(jnp.int32, L)

    def transpose_unit(b):
        src = g_v.at[b]
        dst = o_v.at[b]

        @functools.partial(plsc.parallel_loop, 0, D, unroll=2)
        def dloop(dd):
            d2 = (dd + lanes) & (D - 1)
            dhi = lax.shift_right_logical(d2, 3)
            dlo = d2 & 7
            vals = [
                plsc.load_gather(src, [lanes + L * q, d2]) for q in range(NQ)
            ]
            for q in range(NQ):
                plsc.store_scatter(dst, [dhi, dlo, lanes + L * q], vals[q])

    for i in range(NB - 1):
        gather(i, i).start()

    def outer(p, carry):
        for b in range(NB):
            u = NB * p + b

            @pl.when(p > 0)
            def _():
                write(u - NB, b).wait()
            gather(u, b).wait()
            transpose_unit(b)
            write(u, b).start()

            if b == 0:
                gather(u + NB - 1, (b + NB - 1) % NB).start()
            else:
                @pl.when(p < HIST // NB - 1)
                def _():
                    gather(u + NB - 1, (b + NB - 1) % NB).start()
        return carry

    lax.fori_loop(0, HIST // NB, outer, 0)
    for b in range(NB):
        write(HIST - NB + b, b).wait()


def kernel(token_ids, weight):
    ids_t = token_ids.T.astype(jnp.int32)
    wtail = jnp.pad(weight[NKFULL * 128:].T, ((0, 0), (0, 128 - NTAIL)))
    w2 = _retile(weight.T, wtail)
    table = w2.reshape(NUM_EMBEDDINGS, D)
    out5 = _emb_lookup(ids_t, table)
    return out5.transpose(2, 4, 0, 1, 3).reshape(BATCH, HIST, D)
